# baseline (speedup 1.0000x reference)
import functools

import jax
import jax.numpy as jnp
from jax import lax
from jax.experimental import pallas as pl
from jax.experimental.pallas import tpu as pltpu
from jax.experimental.pallas import tpu_sc as plsc

_L = 16


def _make_emb_kernel(n_rows, d_model, seq_len, n_workers, num_cores):
    rows_per_worker = n_rows // n_workers
    n_chunks = rows_per_worker // seq_len
    n_slices = d_model // _L
    wide = 2 * d_model
    chunk_elems = seq_len * d_model
    n_gath = (seq_len + _L - 1) // _L
    gather_rows = n_gath * _L

    mesh = plsc.VectorSubcoreMesh(core_axis_name="c", subcore_axis_name="s")
    nbuf = 2

    @functools.partial(
        pl.kernel,
        mesh=mesh,
        out_type=jax.ShapeDtypeStruct((n_rows * d_model,), jnp.float32),
        scratch_types=[
            pltpu.VMEM((rows_per_worker + _L,), jnp.int32),
            pltpu.VMEM((rows_per_worker + 2 * _L,), jnp.int32),
            pltpu.VMEM((chunk_elems,), jnp.float32),
            pltpu.VMEM((gather_rows, wide), jnp.float32),
            pltpu.VMEM((gather_rows, wide), jnp.float32),
            pltpu.VMEM((chunk_elems,), jnp.float32),
            pltpu.VMEM((chunk_elems,), jnp.float32),
            pltpu.SemaphoreType.DMA,
            pltpu.SemaphoreType.DMA,
            pltpu.SemaphoreType.DMA,
            pltpu.SemaphoreType.DMA,
        ],
    )
    def emb(ids2_hbm, par_hbm, table_hbm, pos_hbm, out_hbm,
            idx_v, par_v, pos_v, rows0, rows1, st0, st1, g0, g1, s0, s1):
        wid = lax.axis_index("s") * num_cores + lax.axis_index("c")
        base = wid * rows_per_worker
        pltpu.sync_copy(ids2_hbm.at[pl.ds(base, rows_per_worker)],
                        idx_v.at[pl.ds(0, rows_per_worker)])
        idx_v[pl.ds(rows_per_worker, _L)] = jnp.zeros((_L,), jnp.int32)
        pltpu.sync_copy(par_hbm.at[pl.ds(base, rows_per_worker)],
                        par_v.at[pl.ds(0, rows_per_worker)])
        pltpu.sync_copy(pos_hbm.at[pl.ds(0, chunk_elems)], pos_v)

        rows = [rows0, rows1]
        stage = [st0, st1]
        gsems = [g0, g1]
        ssems = [s0, s1]

        def fire_gather(c, slot):
            ioff = pl.multiple_of(c * seq_len, 8)

            @pl.loop(0, n_gath)
            def _(j):
                ivec = idx_v[pl.ds(ioff + _L * j, _L)]
                pltpu.make_async_copy(
                    table_hbm.at[ivec],
                    rows[slot].at[pl.ds(pl.multiple_of(_L * j, 8), _L)],
                    gsems[slot]).start()

        def wait_gather(slot):
            pltpu.make_async_copy(
                table_hbm.at[pl.ds(0, gather_rows)],
                rows[slot],
                gsems[slot]).wait()

        def store_copy(c, slot):
            ooff = pl.multiple_of((base + c * seq_len) * d_model, 8)
            return pltpu.make_async_copy(
                stage[slot],
                out_hbm.at[pl.ds(ooff, chunk_elems)],
                ssems[slot])

        def select_add(c, slot):
            buf = rows[slot]
            st = stage[slot]
            cbase = pl.multiple_of(c * seq_len, 8)

            def body(g, _):
                pvec = par_v[pl.ds(cbase + 8 * g, _L)]
                for r in range(8):
                    i = 8 * g + r
                    p = pvec[r]
                    for k in range(n_slices):
                        o = pl.ds(i * d_model + k * _L, _L)
                        st[o] = buf[i, pl.ds(p + k * _L, _L)] + pos_v[o]
                return 0

            lax.fori_loop(0, seq_len // 8, body, 0)

        def run_chunk(c, slot, nxt, fire_next, wait_store):
            if fire_next:
                fire_gather(c + 1, nxt)
            wait_gather(slot)
            if wait_store == "always":
                store_copy(c - nbuf, slot).wait()
            elif wait_store == "when":
                @pl.when(c >= nbuf)
                def _():
                    store_copy(c - nbuf, slot).wait()
            select_add(c, slot)
            store_copy(c, slot).start()

        fire_gather(0, 0)

        @pl.loop(0, n_chunks - nbuf, step=nbuf)
        def _(g):
            for b in range(nbuf):
                c = g + b
                run_chunk(c, b, (b + 1) % nbuf, True, "when")

        for b in range(nbuf):
            c = n_chunks - nbuf + b
            run_chunk(c, c % nbuf, (c + 1) % nbuf,
                      b + 1 < nbuf, "always")

        for b in range(nbuf):
            store_copy(n_chunks - nbuf + b, (n_chunks - nbuf + b) % nbuf).wait()

    return emb


def kernel(token_ids, token_table, pos_table):
    batch, seq_len = token_ids.shape
    vocab, d_model = token_table.shape
    n_rows = batch * seq_len
    n_workers = 32
    ids_flat = token_ids.reshape(n_rows).astype(jnp.int32)
    ids2 = ids_flat >> 1
    par = (ids_flat & 1) * d_model
    table_wide = token_table.reshape(vocab // 2, 2 * d_model)
    pos_flat = pos_table[:seq_len].reshape(seq_len * d_model)
    emb = _make_emb_kernel(n_rows, d_model, seq_len, n_workers, num_cores=2)
    out_flat = emb(ids2, par, table_wide, pos_flat)
    return out_flat.reshape(batch, seq_len, d_model)


# speedup vs baseline: 1.1294x; 1.1294x over previous
import functools

import jax
import jax.numpy as jnp
from jax import lax
from jax.experimental import pallas as pl
from jax.experimental.pallas import tpu as pltpu
from jax.experimental.pallas import tpu_sc as plsc

_L = 16


def _make_emb_kernel(batch, seq_len, d_model, n_workers, num_cores):
    n_rows = batch * seq_len
    rows_per_worker = n_rows // n_workers
    n_chunks = rows_per_worker // seq_len
    n_slices = d_model // _L
    n_gath = (seq_len + _L - 1) // _L
    gather_rows = n_gath * _L

    mesh = plsc.VectorSubcoreMesh(core_axis_name="c", subcore_axis_name="s")
    nbuf = 2
    row_tiling = (8, 128)

    @functools.partial(
        pl.kernel,
        mesh=mesh,
        out_type=jax.ShapeDtypeStruct((batch * seq_len * d_model,), jnp.float32),
        compiler_params=pltpu.CompilerParams(use_tc_tiling_on_sc=False),
        scratch_types=[
            pltpu.VMEM((rows_per_worker + _L,), jnp.int32),
            pltpu.VMEM((seq_len * d_model,), jnp.float32),
            pltpu.VMEM((gather_rows, d_model), jnp.float32),
            pltpu.VMEM((gather_rows, d_model), jnp.float32),
            pltpu.VMEM((seq_len * d_model,), jnp.float32),
            pltpu.VMEM((seq_len * d_model,), jnp.float32),
            pltpu.SemaphoreType.DMA,
            pltpu.SemaphoreType.DMA,
            pltpu.SemaphoreType.DMA,
            pltpu.SemaphoreType.DMA,
        ],
    )
    def emb(ids_hbm, table_hbm, pos_hbm, out_hbm,
            idx_v, pos_v, rows0, rows1, st0, st1, g0, g1, s0, s1):
        wid = lax.axis_index("s") * num_cores + lax.axis_index("c")
        base = wid * rows_per_worker
        pltpu.sync_copy(ids_hbm.at[pl.ds(base, rows_per_worker)],
                        idx_v.at[pl.ds(0, rows_per_worker)])
        idx_v[pl.ds(rows_per_worker, _L)] = jnp.zeros((_L,), jnp.int32)
        pltpu.sync_copy(pos_hbm.at[pl.ds(0, seq_len * d_model)], pos_v)

        if True:
            rows = [rows0, rows1]
            stage = [st0, st1]
            gsems = [g0, g1]
            ssems = [s0, s1]

            def fire_gather(c, slot):
                ioff = pl.multiple_of(c * seq_len, 8)

                @pl.loop(0, n_gath)
                def _(j):
                    ivec = idx_v[pl.ds(ioff + _L * j, _L)]
                    pltpu.make_async_copy(
                        table_hbm.at[ivec],
                        rows[slot].at[pl.ds(pl.multiple_of(_L * j, 8), _L)],
                        gsems[slot]).start()

            def wait_gather(slot):
                pltpu.make_async_copy(
                    table_hbm.at[pl.ds(0, gather_rows)],
                    rows[slot],
                    gsems[slot]).wait()

            def store_copy(c, slot):
                ooff = pl.multiple_of((base + c * seq_len) * d_model,  8)
                return pltpu.make_async_copy(
                    stage[slot],
                    out_hbm.at[pl.ds(ooff, seq_len * d_model)],
                    ssems[slot])

            def add_pos(slot):
                buf = rows[slot]
                st = stage[slot]

                def body(i, _):
                    for k in range(n_slices):
                        o = pl.ds(i * d_model + k * _L, _L)
                        st[o] = buf[i, pl.ds(k * _L, _L)] + pos_v[o]
                    return 0

                lax.fori_loop(0, seq_len, body, 0)

            def run_chunk(c, slot, nxt, fire_next, wait_store):
                if fire_next:
                    if wait_store == "always":
                        store_copy(c - 1, nxt).wait()
                    elif wait_store == "when":
                        @pl.when(c >= 1)
                        def _():
                            store_copy(c - 1, nxt).wait()
                    fire_gather(c + 1, nxt)
                wait_gather(slot)
                add_pos(slot)
                store_copy(c, slot).start()

            fire_gather(0, 0)

            @pl.loop(0, n_chunks - nbuf, step=nbuf)
            def _(g):
                run_chunk(g, 0, 1, True, "when")
                run_chunk(g + 1, 1, 0, True, "always")

            for b in range(nbuf):
                c = n_chunks - nbuf + b
                run_chunk(c, c % nbuf, (c + 1) % nbuf, b + 1 < nbuf, "always")

            for b in range(nbuf):
                store_copy(n_chunks - nbuf + b, (n_chunks - nbuf + b) % nbuf).wait()


    return emb


def kernel(token_ids, token_table, pos_table):
    batch, seq_len = token_ids.shape
    vocab, d_model = token_table.shape
    n_rows = batch * seq_len
    n_workers = 32
    ids_flat = token_ids.reshape(n_rows).astype(jnp.int32)
    pos_flat = pos_table[:seq_len].reshape(seq_len * d_model)
    emb = _make_emb_kernel(batch, seq_len, d_model, n_workers, num_cores=2)
    out_flat = emb(ids_flat, token_table, pos_flat)
    return out_flat.reshape(batch, seq_len, d_model)
